# baseline (speedup 1.0000x reference)
import functools

import jax
import jax.numpy as jnp
from jax import lax
from jax.experimental import pallas as pl
from jax.experimental.pallas import tpu as pltpu
from jax.experimental.pallas import tpu_sc as plsc

_B = 16384
_D = 64
_NC = 2
_NS = 16
_NW = _NC * _NS
_BPW = _B // _NW
_CHUNK = 128
_NCHUNK = _BPW // _CHUNK
_NSLOT = 3
_L = 16
_TSH = 12


def _perm16(x, idx):
    dnums = lax.GatherDimensionNumbers(
        offset_dims=(), collapsed_slice_dims=(0,), start_index_map=(0,))
    return lax.gather(x, idx[:, None], dnums, slice_sizes=(1,),
                      mode=lax.GatherScatterMode.PROMISE_IN_BOUNDS)


def _hhgr_body(uidx_hbm, iidx_hbm, utab_hbm, itab_hbm, out_hbm,
               uidx_v, iidx_v, urows_v, irows_v, out_v, sems_u, sems_i):
    wid = lax.axis_index("s") * _NC + lax.axis_index("c")
    base = wid * _BPW

    pltpu.sync_copy(uidx_hbm.at[pl.ds(base, _BPW)], uidx_v)
    pltpu.sync_copy(iidx_hbm.at[pl.ds(base, _BPW)], iidx_v)

    def fire_chunk(c, slot):
        def fire(j, carry):
            rb = c * _CHUNK + j * _L
            db = slot * _CHUNK + j * _L
            uvec = uidx_v[pl.ds(rb, _L)]
            ivec = iidx_v[pl.ds(rb, _L)]
            uq = uvec >> 1
            iq = ivec >> 1
            for k in range(_L):
                pltpu.async_copy(utab_hbm.at[uq[k]],
                                 urows_v.at[db + k], sems_u.at[c])
                pltpu.async_copy(itab_hbm.at[iq[k]],
                                 irows_v.at[db + k], sems_i.at[c])
            return carry
        lax.fori_loop(0, _CHUNK // _L, fire, 0)

    lane = lax.iota(jnp.int32, 16)
    perms = [jnp.bitwise_xor(lane, jnp.int32(m)) for m in (1, 2, 4, 8)]

    for c in range(min(_NSLOT, _NCHUNK)):
        fire_chunk(c, c)

    for c in range(_NCHUNK):
        pltpu.make_async_copy(utab_hbm.at[pl.ds(0, _CHUNK)],
                              urows_v.at[pl.ds(0, _CHUNK)], sems_u.at[c]).wait()
        pltpu.make_async_copy(itab_hbm.at[pl.ds(0, _CHUNK)],
                              irows_v.at[pl.ds(0, _CHUNK)], sems_i.at[c]).wait()


        slot = c % _NSLOT

        def grp(g, carry, slot=slot, c=c):
            base_row = slot * _CHUNK + g * _L
            uvec = uidx_v[pl.ds(c * _CHUNK + g * _L, _L)]
            ivec = iidx_v[pl.ds(c * _CHUNK + g * _L, _L)]
            uh = uvec & 1
            ih = ivec & 1
            tot = jnp.zeros((_L,), jnp.float32)
            for r in range(_L):
                urow = urows_v.at[base_row + r]
                irow = irows_v.at[base_row + r]
                uo = uh[r] * _D
                io = ih[r] * _D
                s = urow[pl.ds(uo, 16)] * irow[pl.ds(io, 16)]
                for d in range(1, _D // _L):
                    s = s + (urow[pl.ds(uo + d * 16, 16)] *
                             irow[pl.ds(io + d * 16, 16)])
                for p in perms:
                    s = s + _perm16(s, p)
                tot = jnp.where(lane == r, s, tot)
            out_v[pl.ds(c * _CHUNK + g * _L, _L)] = tot
            return carry

        lax.fori_loop(0, _CHUNK // _L, grp, 0)

        nxt = c + _NSLOT
        if nxt < _NCHUNK:
            fire_chunk(nxt, nxt % _NSLOT)

    pltpu.sync_copy(out_v, out_hbm.at[pl.ds(base, _BPW)])


_hhgr = functools.partial(
    pl.kernel,
    mesh=plsc.VectorSubcoreMesh(core_axis_name="c", subcore_axis_name="s"),
    out_type=jax.ShapeDtypeStruct((_B,), jnp.float32),
    scratch_types=[
        pltpu.VMEM((_BPW,), jnp.int32),
        pltpu.VMEM((_BPW,), jnp.int32),
        pltpu.VMEM((_NSLOT * _CHUNK, 2 * _D), jnp.float32),
        pltpu.VMEM((_NSLOT * _CHUNK, 2 * _D), jnp.float32),
        pltpu.VMEM((_BPW,), jnp.float32),
        pltpu.SemaphoreType.DMA((_NCHUNK,)),
        pltpu.SemaphoreType.DMA((_NCHUNK,)),
    ],
)(_hhgr_body)


def kernel(user_inputs, item_inputs, user_table, item_table):
    ut = user_table.reshape(500000, 2 * _D)
    it = item_table.reshape(500000, 2 * _D)
    return _hhgr(user_inputs.astype(jnp.int32), item_inputs.astype(jnp.int32),
                 ut, it)


# speedup vs baseline: 2.0633x; 2.0633x over previous
import functools

import jax
import jax.numpy as jnp
from jax import lax
from jax.experimental import pallas as pl
from jax.experimental.pallas import tpu as pltpu
from jax.experimental.pallas import tpu_sc as plsc

_B = 16384
_D = 64
_NC = 2
_NS = 16
_NW = _NC * _NS
_BPW = _B // _NW
_CHUNK = 128
_NCHUNK = _BPW // _CHUNK
_NSLOT = 3
_L = 16
_TSH = 13


def _perm16(x, idx):
    dnums = lax.GatherDimensionNumbers(
        offset_dims=(), collapsed_slice_dims=(0,), start_index_map=(0,))
    return lax.gather(x, idx[:, None], dnums, slice_sizes=(1,),
                      mode=lax.GatherScatterMode.PROMISE_IN_BOUNDS)


def _hhgr_body(uidx_hbm, iidx_hbm, utab_hbm, itab_hbm, out_hbm,
               uidx_v, iidx_v, urows_v, irows_v, out_v, sems_u, sems_i):
    wid = lax.axis_index("s") * _NC + lax.axis_index("c")
    base = wid * _BPW

    pltpu.sync_copy(uidx_hbm.at[pl.ds(base, _BPW)], uidx_v)
    pltpu.sync_copy(iidx_hbm.at[pl.ds(base, _BPW)], iidx_v)

    def fire_chunk(c, slot):
        def fire(j, carry):
            rb = c * _CHUNK + j * _L
            db = slot * _CHUNK + j * _L
            uvec = uidx_v[pl.ds(rb, _L)]
            ivec = iidx_v[pl.ds(rb, _L)]
            uq = ((uvec >> _TSH) * _TH) + (uvec & (_TH - 1))
            iq = ((ivec >> _TSH) * _TH) + (ivec & (_TH - 1))
            for k in range(_L):
                pltpu.async_copy(utab_hbm.at[uq[k]],
                                 urows_v.at[db + k], sems_u.at[c])
                pltpu.async_copy(itab_hbm.at[iq[k]],
                                 irows_v.at[db + k], sems_i.at[c])
            return carry
        lax.fori_loop(0, _CHUNK // _L, fire, 0)

    lane = lax.iota(jnp.int32, 16)
    perms = [jnp.bitwise_xor(lane, jnp.int32(m)) for m in (1, 2, 4, 8)]

    for c in range(min(_NSLOT, _NCHUNK)):
        fire_chunk(c, c)

    for c in range(_NCHUNK):
        pltpu.make_async_copy(utab_hbm.at[pl.ds(0, _CHUNK)],
                              urows_v.at[pl.ds(0, _CHUNK)], sems_u.at[c]).wait()
        pltpu.make_async_copy(itab_hbm.at[pl.ds(0, _CHUNK)],
                              irows_v.at[pl.ds(0, _CHUNK)], sems_i.at[c]).wait()


        slot = c % _NSLOT

        def grp(g, carry, slot=slot, c=c):
            base_row = slot * _CHUNK + g * _L
            uvec = uidx_v[pl.ds(c * _CHUNK + g * _L, _L)]
            ivec = iidx_v[pl.ds(c * _CHUNK + g * _L, _L)]
            uh = (uvec >> (_TSH - 1)) & 1
            ih = (ivec >> (_TSH - 1)) & 1
            tot = jnp.zeros((_L,), jnp.float32)
            for r in range(_L):
                urow = urows_v.at[base_row + r]
                irow = irows_v.at[base_row + r]
                uo = uh[r] * _D
                io = ih[r] * _D
                s = urow[pl.ds(uo, 16)] * irow[pl.ds(io, 16)]
                for d in range(1, _D // _L):
                    s = s + (urow[pl.ds(uo + d * 16, 16)] *
                             irow[pl.ds(io + d * 16, 16)])
                for p in perms:
                    s = s + _perm16(s, p)
                tot = jnp.where(lane == r, s, tot)
            out_v[pl.ds(c * _CHUNK + g * _L, _L)] = tot
            return carry

        lax.fori_loop(0, _CHUNK // _L, grp, 0)

        nxt = c + _NSLOT
        if nxt < _NCHUNK:
            fire_chunk(nxt, nxt % _NSLOT)

    pltpu.sync_copy(out_v, out_hbm.at[pl.ds(base, _BPW)])


_hhgr = functools.partial(
    pl.kernel,
    mesh=plsc.VectorSubcoreMesh(core_axis_name="c", subcore_axis_name="s"),
    out_type=jax.ShapeDtypeStruct((_B,), jnp.float32),
    scratch_types=[
        pltpu.VMEM((_BPW,), jnp.int32),
        pltpu.VMEM((_BPW,), jnp.int32),
        pltpu.VMEM((_NSLOT * _CHUNK, 2 * _D), jnp.float32),
        pltpu.VMEM((_NSLOT * _CHUNK, 2 * _D), jnp.float32),
        pltpu.VMEM((_BPW,), jnp.float32),
        pltpu.SemaphoreType.DMA((_NCHUNK,)),
        pltpu.SemaphoreType.DMA((_NCHUNK,)),
    ],
)(_hhgr_body)


_TBLK = 8192
_TH = _TBLK // 2


def _tc_transpose_body(tT_ref, o_ref):
    eye = jnp.eye(_D, dtype=jnp.float32)
    o_ref[:, 0:_D] = lax.dot_general(
        tT_ref[:, 0:_TH], eye, (((0,), (0,)), ((), ())),
        preferred_element_type=jnp.float32)
    o_ref[:, _D:2 * _D] = lax.dot_general(
        tT_ref[:, _TH:_TBLK], eye, (((0,), (0,)), ((), ())),
        preferred_element_type=jnp.float32)


def _tc_transpose(tabT):
    n = tabT.shape[1]
    grid = (n + _TBLK - 1) // _TBLK
    return pl.pallas_call(
        _tc_transpose_body,
        grid=(grid,),
        in_specs=[pl.BlockSpec((_D, _TBLK), lambda i: (0, i))],
        out_specs=pl.BlockSpec((_TH, 2 * _D), lambda i: (i, 0)),
        out_shape=jax.ShapeDtypeStruct((grid * _TH, 2 * _D), jnp.float32),
    )(tabT)


def kernel(user_inputs, item_inputs, user_table, item_table):
    ut = _tc_transpose(user_table.T)
    it = _tc_transpose(item_table.T)
    return _hhgr(user_inputs.astype(jnp.int32), item_inputs.astype(jnp.int32),
                 ut, it)


# speedup vs baseline: 2.3461x; 1.1371x over previous
import functools

import jax
import jax.numpy as jnp
from jax import lax
from jax.experimental import pallas as pl
from jax.experimental.pallas import tpu as pltpu
from jax.experimental.pallas import tpu_sc as plsc

_B = 16384
_D = 64
_NC = 2
_NS = 16
_NW = _NC * _NS
_BPW = _B // _NW
_CHUNK = 128
_NCHUNK = _BPW // _CHUNK
_NSLOT = 3
_L = 16
_TSH = 14


def _perm16(x, idx):
    dnums = lax.GatherDimensionNumbers(
        offset_dims=(), collapsed_slice_dims=(0,), start_index_map=(0,))
    return lax.gather(x, idx[:, None], dnums, slice_sizes=(1,),
                      mode=lax.GatherScatterMode.PROMISE_IN_BOUNDS)


def _hhgr_body(uidx_hbm, iidx_hbm, utab_hbm, itab_hbm, out_hbm,
               uidx_v, iidx_v, urows_v, irows_v, out_v, sems_u, sems_i):
    wid = lax.axis_index("s") * _NC + lax.axis_index("c")
    base = wid * _BPW

    pltpu.sync_copy(uidx_hbm.at[pl.ds(base, _BPW)], uidx_v)
    pltpu.sync_copy(iidx_hbm.at[pl.ds(base, _BPW)], iidx_v)

    def fire_chunk(c, slot):
        def fire(j, carry):
            rb = c * _CHUNK + j * _L
            db = slot * _CHUNK + j * _L
            uvec = uidx_v[pl.ds(rb, _L)]
            ivec = iidx_v[pl.ds(rb, _L)]
            uq = ((uvec >> _TSH) * _TH) + (uvec & (_TH - 1))
            iq = ((ivec >> _TSH) * _TH) + (ivec & (_TH - 1))
            for k in range(_L):
                pltpu.async_copy(utab_hbm.at[uq[k]],
                                 urows_v.at[db + k], sems_u.at[c])
                pltpu.async_copy(itab_hbm.at[iq[k]],
                                 irows_v.at[db + k], sems_i.at[c])
            return carry
        lax.fori_loop(0, _CHUNK // _L, fire, 0)

    lane = lax.iota(jnp.int32, 16)
    perms = [jnp.bitwise_xor(lane, jnp.int32(m)) for m in (1, 2, 4, 8)]

    for c in range(min(_NSLOT, _NCHUNK)):
        fire_chunk(c, c)

    for c in range(_NCHUNK):
        pltpu.make_async_copy(utab_hbm.at[pl.ds(0, _CHUNK)],
                              urows_v.at[pl.ds(0, _CHUNK)], sems_u.at[c]).wait()
        pltpu.make_async_copy(itab_hbm.at[pl.ds(0, _CHUNK)],
                              irows_v.at[pl.ds(0, _CHUNK)], sems_i.at[c]).wait()


        slot = c % _NSLOT

        def grp(g, carry, slot=slot, c=c):
            base_row = slot * _CHUNK + g * _L
            uvec = uidx_v[pl.ds(c * _CHUNK + g * _L, _L)]
            ivec = iidx_v[pl.ds(c * _CHUNK + g * _L, _L)]
            uh = (uvec >> (_TSH - 1)) & 1
            ih = (ivec >> (_TSH - 1)) & 1
            tot = jnp.zeros((_L,), jnp.float32)
            for r in range(_L):
                urow = urows_v.at[base_row + r]
                irow = irows_v.at[base_row + r]
                uo = uh[r] * _D
                io = ih[r] * _D
                s = urow[pl.ds(uo, 16)] * irow[pl.ds(io, 16)]
                for d in range(1, _D // _L):
                    s = s + (urow[pl.ds(uo + d * 16, 16)] *
                             irow[pl.ds(io + d * 16, 16)])
                for p in perms:
                    s = s + _perm16(s, p)
                tot = jnp.where(lane == r, s, tot)
            out_v[pl.ds(c * _CHUNK + g * _L, _L)] = tot
            return carry

        lax.fori_loop(0, _CHUNK // _L, grp, 0)

        nxt = c + _NSLOT
        if nxt < _NCHUNK:
            fire_chunk(nxt, nxt % _NSLOT)

    pltpu.sync_copy(out_v, out_hbm.at[pl.ds(base, _BPW)])


_hhgr = functools.partial(
    pl.kernel,
    mesh=plsc.VectorSubcoreMesh(core_axis_name="c", subcore_axis_name="s"),
    out_type=jax.ShapeDtypeStruct((_B,), jnp.float32),
    scratch_types=[
        pltpu.VMEM((_BPW,), jnp.int32),
        pltpu.VMEM((_BPW,), jnp.int32),
        pltpu.VMEM((_NSLOT * _CHUNK, 2 * _D), jnp.float32),
        pltpu.VMEM((_NSLOT * _CHUNK, 2 * _D), jnp.float32),
        pltpu.VMEM((_BPW,), jnp.float32),
        pltpu.SemaphoreType.DMA((_NCHUNK,)),
        pltpu.SemaphoreType.DMA((_NCHUNK,)),
    ],
)(_hhgr_body)


_TBLK = 16384
_TH = _TBLK // 2


def _tc_transpose_body(tT_ref, o_ref):
    eye = jnp.eye(_D, dtype=jnp.float32)
    o_ref[:, 0:_D] = lax.dot_general(
        tT_ref[:, 0:_TH], eye, (((0,), (0,)), ((), ())),
        preferred_element_type=jnp.float32)
    o_ref[:, _D:2 * _D] = lax.dot_general(
        tT_ref[:, _TH:_TBLK], eye, (((0,), (0,)), ((), ())),
        preferred_element_type=jnp.float32)


def _tc_transpose(tabT):
    n = tabT.shape[1]
    grid = (n + _TBLK - 1) // _TBLK
    return pl.pallas_call(
        _tc_transpose_body,
        grid=(grid,),
        in_specs=[pl.BlockSpec((_D, _TBLK), lambda i: (0, i))],
        out_specs=pl.BlockSpec((_TH, 2 * _D), lambda i: (i, 0)),
        out_shape=jax.ShapeDtypeStruct((grid * _TH, 2 * _D), jnp.float32),
    )(tabT)


def kernel(user_inputs, item_inputs, user_table, item_table):
    ut = _tc_transpose(user_table.T)
    it = _tc_transpose(item_table.T)
    return _hhgr(user_inputs.astype(jnp.int32), item_inputs.astype(jnp.int32),
                 ut, it)


# speedup vs baseline: 2.4919x; 1.0621x over previous
import functools

import jax
import jax.numpy as jnp
from jax import lax
from jax.experimental import pallas as pl
from jax.experimental.pallas import tpu as pltpu
from jax.experimental.pallas import tpu_sc as plsc

_B = 16384
_D = 64
_NC = 2
_NS = 16
_NW = _NC * _NS
_BPW = _B // _NW
_CHUNK = 128
_NCHUNK = _BPW // _CHUNK
_NSLOT = 3
_L = 16
_TSH = 15


def _perm16(x, idx):
    dnums = lax.GatherDimensionNumbers(
        offset_dims=(), collapsed_slice_dims=(0,), start_index_map=(0,))
    return lax.gather(x, idx[:, None], dnums, slice_sizes=(1,),
                      mode=lax.GatherScatterMode.PROMISE_IN_BOUNDS)


def _hhgr_body(uidx_hbm, iidx_hbm, utab_hbm, itab_hbm, out_hbm,
               uidx_v, iidx_v, urows_v, irows_v, out_v, sems_u, sems_i):
    wid = lax.axis_index("s") * _NC + lax.axis_index("c")
    base = wid * _BPW

    pltpu.sync_copy(uidx_hbm.at[pl.ds(base, _BPW)], uidx_v)
    pltpu.sync_copy(iidx_hbm.at[pl.ds(base, _BPW)], iidx_v)

    def fire_chunk(c, slot):
        def fire(j, carry):
            rb = c * _CHUNK + j * _L
            db = slot * _CHUNK + j * _L
            uvec = uidx_v[pl.ds(rb, _L)]
            ivec = iidx_v[pl.ds(rb, _L)]
            uq = ((uvec >> _TSH) * _TH) + (uvec & (_TH - 1))
            iq = ((ivec >> _TSH) * _TH) + (ivec & (_TH - 1))
            for k in range(_L):
                pltpu.async_copy(utab_hbm.at[uq[k]],
                                 urows_v.at[db + k], sems_u.at[c])
                pltpu.async_copy(itab_hbm.at[iq[k]],
                                 irows_v.at[db + k], sems_i.at[c])
            return carry
        lax.fori_loop(0, _CHUNK // _L, fire, 0)

    lane = lax.iota(jnp.int32, 16)
    perms = [jnp.bitwise_xor(lane, jnp.int32(m)) for m in (1, 2, 4, 8)]

    for c in range(min(_NSLOT, _NCHUNK)):
        fire_chunk(c, c)

    for c in range(_NCHUNK):
        pltpu.make_async_copy(utab_hbm.at[pl.ds(0, _CHUNK)],
                              urows_v.at[pl.ds(0, _CHUNK)], sems_u.at[c]).wait()
        pltpu.make_async_copy(itab_hbm.at[pl.ds(0, _CHUNK)],
                              irows_v.at[pl.ds(0, _CHUNK)], sems_i.at[c]).wait()


        slot = c % _NSLOT

        def grp(g, carry, slot=slot, c=c):
            base_row = slot * _CHUNK + g * _L
            uvec = uidx_v[pl.ds(c * _CHUNK + g * _L, _L)]
            ivec = iidx_v[pl.ds(c * _CHUNK + g * _L, _L)]
            uh = (uvec >> (_TSH - 1)) & 1
            ih = (ivec >> (_TSH - 1)) & 1
            tot = jnp.zeros((_L,), jnp.float32)
            for r in range(_L):
                urow = urows_v.at[base_row + r]
                irow = irows_v.at[base_row + r]
                uo = uh[r] * _D
                io = ih[r] * _D
                s = urow[pl.ds(uo, 16)] * irow[pl.ds(io, 16)]
                for d in range(1, _D // _L):
                    s = s + (urow[pl.ds(uo + d * 16, 16)] *
                             irow[pl.ds(io + d * 16, 16)])
                for p in perms:
                    s = s + _perm16(s, p)
                tot = jnp.where(lane == r, s, tot)
            out_v[pl.ds(c * _CHUNK + g * _L, _L)] = tot
            return carry

        lax.fori_loop(0, _CHUNK // _L, grp, 0)

        nxt = c + _NSLOT
        if nxt < _NCHUNK:
            fire_chunk(nxt, nxt % _NSLOT)

    pltpu.sync_copy(out_v, out_hbm.at[pl.ds(base, _BPW)])


_hhgr = functools.partial(
    pl.kernel,
    mesh=plsc.VectorSubcoreMesh(core_axis_name="c", subcore_axis_name="s"),
    out_type=jax.ShapeDtypeStruct((_B,), jnp.float32),
    scratch_types=[
        pltpu.VMEM((_BPW,), jnp.int32),
        pltpu.VMEM((_BPW,), jnp.int32),
        pltpu.VMEM((_NSLOT * _CHUNK, 2 * _D), jnp.float32),
        pltpu.VMEM((_NSLOT * _CHUNK, 2 * _D), jnp.float32),
        pltpu.VMEM((_BPW,), jnp.float32),
        pltpu.SemaphoreType.DMA((_NCHUNK,)),
        pltpu.SemaphoreType.DMA((_NCHUNK,)),
    ],
)(_hhgr_body)


_TBLK = 32768
_TH = _TBLK // 2


def _tc_transpose_body(tT_ref, o_ref):
    eye = jnp.eye(_D, dtype=jnp.float32)
    o_ref[:, 0:_D] = lax.dot_general(
        tT_ref[:, 0:_TH], eye, (((0,), (0,)), ((), ())),
        preferred_element_type=jnp.float32)
    o_ref[:, _D:2 * _D] = lax.dot_general(
        tT_ref[:, _TH:_TBLK], eye, (((0,), (0,)), ((), ())),
        preferred_element_type=jnp.float32)


def _tc_transpose(tabT):
    n = tabT.shape[1]
    grid = (n + _TBLK - 1) // _TBLK
    return pl.pallas_call(
        _tc_transpose_body,
        grid=(grid,),
        in_specs=[pl.BlockSpec((_D, _TBLK), lambda i: (0, i))],
        out_specs=pl.BlockSpec((_TH, 2 * _D), lambda i: (i, 0)),
        out_shape=jax.ShapeDtypeStruct((grid * _TH, 2 * _D), jnp.float32),
    )(tabT)


def kernel(user_inputs, item_inputs, user_table, item_table):
    ut = _tc_transpose(user_table.T)
    it = _tc_transpose(item_table.T)
    return _hhgr(user_inputs.astype(jnp.int32), item_inputs.astype(jnp.int32),
                 ut, it)
